# baseline (speedup 1.0000x reference)
import jax
import jax.numpy as jnp
from jax import lax
from jax.experimental import pallas as pl
from jax.experimental.pallas import tpu as pltpu

T = 12
N = 307
C = 64
B = 8
NP = 384
BC = B * C
NW = T - 2
NG = 3
NWS = 5

_TDIMS = (((1,), (1,)), ((), ()))


def _body(xr, tec, seb, ap, wt, bc, out, xe):
    f32 = jnp.float32
    i = pl.program_id(0)

    def fill(t):
        for bi in range(B):
            slab = jnp.transpose(xr[bi, t])
            tecb = tec[t, bi * C:(bi + 1) * C]
            sebb = seb[bi * C:(bi + 1) * C, :N]
            v = jnp.pad(slab + tecb + sebb, ((0, 0), (0, NP - N)))
            xe[t, bi * C:(bi + 1) * C, :] = v

    @pl.when(i == 0)
    def _():
        fill(0)
        fill(1)
    for k in range(NWS):
        fill(NWS * i + 2 + k)

    def glu(y, wtj, bcj):
        parts = []
        for bi in range(B):
            yb = y[bi * C:(bi + 1) * C, :]
            t = jnp.dot(wtj, yb, preferred_element_type=f32) + bcj
            parts.append(t[:C] * jax.nn.sigmoid(t[C:]))
        return jnp.concatenate(parts, axis=0)

    def window(w, slot):
        X0 = xe[NWS * i + slot]
        X1 = xe[NWS * i + slot + 1]
        X2 = xe[NWS * i + slot + 2]
        h = None
        acc = None
        for j in range(NG):
            wtj = jnp.transpose(wt[slot * NG + j])
            bcj = bc[slot * NG + j]
            if j == 0:
                y = (lax.dot_general(X0, ap[:, 0:NP], _TDIMS, preferred_element_type=f32)
                     + lax.dot_general(X1, ap[:, NP:2 * NP], _TDIMS, preferred_element_type=f32)
                     + lax.dot_general(X2, ap[:, 2 * NP:3 * NP], _TDIMS, preferred_element_type=f32))
            elif j == 1:
                y = lax.dot_general(h, ap[...], _TDIMS, preferred_element_type=f32)
            else:
                y = lax.dot_general(h, ap[NP:2 * NP, :], _TDIMS, preferred_element_type=f32)
            g = glu(y, wtj, bcj)
            if j < NG - 1:
                h = g
                c = g[:, NP:2 * NP]
            else:
                c = g
            acc = c if acc is None else jnp.maximum(acc, c)
        for bi in range(B):
            tb = jnp.transpose(acc[bi * C:(bi + 1) * C, :])
            out[bi, slot] = tb[:N, :]

    for k in range(NWS):
        window(NWS * i + k, k)


def kernel(x, A, temporal_emb, spatial_emb, W, b):
    tec = jnp.tile(temporal_emb.reshape(T, C), (1, B)).reshape(T, BC, 1)
    seb = jnp.tile(spatial_emb.reshape(N, C).T, (B, 1))
    seb = jnp.pad(seb, ((0, 0), (0, NP - N)))

    A4 = A.reshape(3, N, 3, N)
    Ap = jnp.pad(A4, ((0, 0), (0, NP - N), (0, 0), (0, NP - N)))
    Ap = Ap.reshape(3 * NP, 3 * NP)

    bcol = b.reshape(NW * NG, 2 * C, 1)

    full = lambda shape: pl.BlockSpec(shape, lambda i: (0,) * len(shape))

    out = pl.pallas_call(
        _body,
        grid=(NW // NWS,),
        in_specs=[
            full((B, T, N, C)),
            full((T, BC, 1)),
            full((BC, NP)),
            full((3 * NP, 3 * NP)),
            pl.BlockSpec((NWS * NG, C, 2 * C), lambda i: (i, 0, 0)),
            pl.BlockSpec((NWS * NG, 2 * C, 1), lambda i: (i, 0, 0)),
        ],
        out_specs=pl.BlockSpec((B, NWS, N, C), lambda i: (0, i, 0, 0)),
        out_shape=jax.ShapeDtypeStruct((B, NW, N, C), jnp.float32),
        scratch_shapes=[pltpu.VMEM((T, BC, NP), jnp.float32)],
    )(x, tec, seb, Ap, W, bcol)

    return out


# speedup vs baseline: 1.1807x; 1.1807x over previous
import jax
import jax.numpy as jnp
from jax import lax
from jax.experimental import pallas as pl
from jax.experimental.pallas import tpu as pltpu

T = 12
N = 307
C = 64
B = 8
NP = 384
BC = B * C
NW = T - 2
NG = 3

_TDIMS = (((1,), (1,)), ((), ()))


def _body(xr, tec, seb, ap, wt, bc, out, xe):
    f32 = jnp.float32
    i = pl.program_id(0)

    def fill(t):
        for bi in range(B):
            slab = jnp.transpose(xr[bi, t])
            tecb = tec[t, bi * C:(bi + 1) * C]
            sebb = seb[bi * C:(bi + 1) * C, :N]
            v = jnp.pad(slab + tecb + sebb, ((0, 0), (0, NP - N)))
            xe[t, bi * C:(bi + 1) * C, :] = v

    @pl.when(i == 0)
    def _():
        fill(0)
        fill(1)
    fill(2 * i + 2)
    fill(2 * i + 3)

    def glu(y, wtj, bcj):
        parts = []
        for bi in range(B):
            yb = y[bi * C:(bi + 1) * C, :]
            t = jnp.dot(wtj, yb, preferred_element_type=f32) + bcj
            parts.append(t[:C] * jax.nn.sigmoid(t[C:]))
        return jnp.concatenate(parts, axis=0)

    def window(w, slot):
        X0 = xe[2 * i + slot]
        X1 = xe[2 * i + slot + 1]
        X2 = xe[2 * i + slot + 2]
        h = None
        acc = None
        for j in range(NG):
            wtj = jnp.transpose(wt[slot * NG + j])
            bcj = bc[slot * NG + j]
            if j == 0:
                y = (lax.dot_general(X0, ap[:, 0:NP], _TDIMS, preferred_element_type=f32)
                     + lax.dot_general(X1, ap[:, NP:2 * NP], _TDIMS, preferred_element_type=f32)
                     + lax.dot_general(X2, ap[:, 2 * NP:3 * NP], _TDIMS, preferred_element_type=f32))
            elif j == 1:
                y = lax.dot_general(h, ap[...], _TDIMS, preferred_element_type=f32)
            else:
                y = lax.dot_general(h, ap[NP:2 * NP, :], _TDIMS, preferred_element_type=f32)
            g = glu(y, wtj, bcj)
            if j < NG - 1:
                h = g
                c = g[:, NP:2 * NP]
            else:
                c = g
            acc = c if acc is None else jnp.maximum(acc, c)
        for bi in range(B):
            tb = jnp.transpose(acc[bi * C:(bi + 1) * C, :])
            out[bi, slot] = tb[:N, :]

    window(2 * i, 0)
    window(2 * i + 1, 1)


def kernel(x, A, temporal_emb, spatial_emb, W, b):
    tec = jnp.tile(temporal_emb.reshape(T, C), (1, B)).reshape(T, BC, 1)
    seb = jnp.tile(spatial_emb.reshape(N, C).T, (B, 1))
    seb = jnp.pad(seb, ((0, 0), (0, NP - N)))

    A4 = A.reshape(3, N, 3, N)
    Ap = jnp.pad(A4, ((0, 0), (0, NP - N), (0, 0), (0, NP - N)))
    Ap = Ap.reshape(3 * NP, 3 * NP)

    bcol = b.reshape(NW * NG, 2 * C, 1)

    full = lambda shape: pl.BlockSpec(shape, lambda i: (0,) * len(shape))

    out = pl.pallas_call(
        _body,
        grid=(NW // 2,),
        in_specs=[
            full((B, T, N, C)),
            full((T, BC, 1)),
            full((BC, NP)),
            full((3 * NP, 3 * NP)),
            pl.BlockSpec((2 * NG, C, 2 * C), lambda i: (i, 0, 0)),
            pl.BlockSpec((2 * NG, 2 * C, 1), lambda i: (i, 0, 0)),
        ],
        out_specs=pl.BlockSpec((B, 2, N, C), lambda i: (0, i, 0, 0)),
        out_shape=jax.ShapeDtypeStruct((B, NW, N, C), jnp.float32),
        scratch_shapes=[pltpu.VMEM((T, BC, NP), jnp.float32)],
    )(x, tec, seb, Ap, W, bcol)

    return out


# speedup vs baseline: 1.3467x; 1.1406x over previous
import jax
import jax.numpy as jnp
from jax import lax
from jax.experimental import pallas as pl
from jax.experimental.pallas import tpu as pltpu

T = 12
N = 307
C = 64
B = 8
NP = 384
BC = B * C
NW = T - 2
NG = 3

_TDIMS = (((1,), (1,)), ((), ()))


def _body(xr, tec, seb, ap, wt, bc, out, xe):
    f32 = jnp.float32
    i = pl.program_id(0)

    def fill(t):
        for bi in range(B):
            slab = xr[bi, t]
            tecb = tec[t, bi * C:(bi + 1) * C]
            sebb = seb[bi * C:(bi + 1) * C, :N]
            v = jnp.pad(slab + tecb + sebb, ((0, 0), (0, NP - N)))
            xe[t, bi * C:(bi + 1) * C, :] = v

    @pl.when(i == 0)
    def _():
        fill(0)
        fill(1)
    fill(2 * i + 2)
    fill(2 * i + 3)

    def glu(y, wtj, bcj):
        parts = []
        for bi in range(B):
            yb = y[bi * C:(bi + 1) * C, :]
            t = jnp.dot(wtj, yb, preferred_element_type=f32) + bcj
            parts.append(t[:C] * jax.nn.sigmoid(t[C:]))
        return jnp.concatenate(parts, axis=0)

    def window(w, slot):
        X0 = xe[2 * i + slot]
        X1 = xe[2 * i + slot + 1]
        X2 = xe[2 * i + slot + 2]
        h = None
        acc = None
        for j in range(NG):
            wtj = jnp.transpose(wt[slot * NG + j])
            bcj = bc[slot * NG + j]
            if j == 0:
                y = (lax.dot_general(X0, ap[:, 0:NP], _TDIMS, preferred_element_type=f32)
                     + lax.dot_general(X1, ap[:, NP:2 * NP], _TDIMS, preferred_element_type=f32)
                     + lax.dot_general(X2, ap[:, 2 * NP:3 * NP], _TDIMS, preferred_element_type=f32))
            elif j == 1:
                y = lax.dot_general(h, ap[...], _TDIMS, preferred_element_type=f32)
            else:
                y = lax.dot_general(h, ap[NP:2 * NP, :], _TDIMS, preferred_element_type=f32)
            g = glu(y, wtj, bcj)
            if j < NG - 1:
                h = g
                c = g[:, NP:2 * NP]
            else:
                c = g
            acc = c if acc is None else jnp.maximum(acc, c)
        for bi in range(B):
            tb = jnp.transpose(acc[bi * C:(bi + 1) * C, :])
            out[bi, slot] = tb[:N, :]

    window(2 * i, 0)
    window(2 * i + 1, 1)


def kernel(x, A, temporal_emb, spatial_emb, W, b):
    xt = jnp.transpose(x, (0, 1, 3, 2))

    tec = jnp.tile(temporal_emb.reshape(T, C), (1, B)).reshape(T, BC, 1)
    seb = jnp.tile(spatial_emb.reshape(N, C).T, (B, 1))
    seb = jnp.pad(seb, ((0, 0), (0, NP - N)))

    A4 = A.reshape(3, N, 3, N)
    Ap = jnp.pad(A4, ((0, 0), (0, NP - N), (0, 0), (0, NP - N)))
    Ap = Ap.reshape(3 * NP, 3 * NP)

    bcol = b.reshape(NW * NG, 2 * C, 1)

    full = lambda shape: pl.BlockSpec(shape, lambda i: (0,) * len(shape))

    out = pl.pallas_call(
        _body,
        grid=(NW // 2,),
        in_specs=[
            full((B, T, C, N)),
            full((T, BC, 1)),
            full((BC, NP)),
            full((3 * NP, 3 * NP)),
            pl.BlockSpec((2 * NG, C, 2 * C), lambda i: (i, 0, 0)),
            pl.BlockSpec((2 * NG, 2 * C, 1), lambda i: (i, 0, 0)),
        ],
        out_specs=pl.BlockSpec((B, 2, N, C), lambda i: (0, i, 0, 0)),
        out_shape=jax.ShapeDtypeStruct((B, NW, N, C), jnp.float32),
        scratch_shapes=[pltpu.VMEM((T, BC, NP), jnp.float32)],
    )(xt, tec, seb, Ap, W, bcol)

    return out


# speedup vs baseline: 1.5248x; 1.1322x over previous
import jax
import jax.numpy as jnp
from jax import lax
from jax.experimental import pallas as pl
from jax.experimental.pallas import tpu as pltpu

T = 12
N = 307
C = 64
B = 8
NP = 384
BC = B * C
NW = T - 2
NG = 3

_TDIMS = (((1,), (1,)), ((), ()))


def _body(xr, tec, seb, ap, wt, bc, out, xe):
    f32 = jnp.float32
    i = pl.program_id(0)

    def fill(t):
        for bi in range(B):
            slab = xr[bi, t]
            tecb = tec[t, bi * C:(bi + 1) * C]
            sebb = seb[bi * C:(bi + 1) * C, :N]
            v = jnp.pad(slab + tecb + sebb, ((0, 0), (0, NP - N)))
            xe[t, bi * C:(bi + 1) * C, :] = v

    @pl.when(i == 0)
    def _():
        fill(0)
        fill(1)
    fill(2 * i + 2)
    fill(2 * i + 3)

    def glu(y, wtj, bcj):
        parts = []
        for bi in range(B):
            yb = y[bi * C:(bi + 1) * C, :]
            t = jnp.dot(wtj, yb, preferred_element_type=f32) + bcj
            parts.append(t[:C] * jax.nn.sigmoid(t[C:]))
        return jnp.concatenate(parts, axis=0)

    def window(w, slot):
        X0 = xe[2 * i + slot]
        X1 = xe[2 * i + slot + 1]
        X2 = xe[2 * i + slot + 2]
        h = None
        acc = None
        for j in range(NG):
            wtj = jnp.transpose(wt[slot * NG + j])
            bcj = bc[slot * NG + j]
            if j == 0:
                y = (lax.dot_general(X0, ap[:, 0:NP], _TDIMS, preferred_element_type=f32)
                     + lax.dot_general(X1, ap[:, NP:2 * NP], _TDIMS, preferred_element_type=f32)
                     + lax.dot_general(X2, ap[:, 2 * NP:3 * NP], _TDIMS, preferred_element_type=f32))
            elif j == 1:
                y = lax.dot_general(h, ap[...], _TDIMS, preferred_element_type=f32)
            else:
                y = lax.dot_general(h, ap[NP:2 * NP, :], _TDIMS, preferred_element_type=f32)
            g = glu(y, wtj, bcj)
            if j < NG - 1:
                h = g
                c = g[:, NP:2 * NP]
            else:
                c = g
            acc = c if acc is None else jnp.maximum(acc, c)
        for bi in range(B):
            out[bi, slot] = acc[bi * C:(bi + 1) * C, 0:N]

    window(2 * i, 0)
    window(2 * i + 1, 1)


def kernel(x, A, temporal_emb, spatial_emb, W, b):
    xt = jnp.transpose(x, (0, 1, 3, 2))

    tec = jnp.tile(temporal_emb.reshape(T, C), (1, B)).reshape(T, BC, 1)
    seb = jnp.tile(spatial_emb.reshape(N, C).T, (B, 1))
    seb = jnp.pad(seb, ((0, 0), (0, NP - N)))

    A4 = A.reshape(3, N, 3, N)
    Ap = jnp.pad(A4, ((0, 0), (0, NP - N), (0, 0), (0, NP - N)))
    Ap = Ap.reshape(3 * NP, 3 * NP)

    bcol = b.reshape(NW * NG, 2 * C, 1)

    full = lambda shape: pl.BlockSpec(shape, lambda i: (0,) * len(shape))

    out = pl.pallas_call(
        _body,
        grid=(NW // 2,),
        in_specs=[
            full((B, T, C, N)),
            full((T, BC, 1)),
            full((BC, NP)),
            full((3 * NP, 3 * NP)),
            pl.BlockSpec((2 * NG, C, 2 * C), lambda i: (i, 0, 0)),
            pl.BlockSpec((2 * NG, 2 * C, 1), lambda i: (i, 0, 0)),
        ],
        out_specs=pl.BlockSpec((B, 2, C, N), lambda i: (0, i, 0, 0)),
        out_shape=jax.ShapeDtypeStruct((B, NW, C, N), jnp.float32),
        scratch_shapes=[pltpu.VMEM((T, BC, NP), jnp.float32)],
    )(xt, tec, seb, Ap, W, bcol)

    return jnp.transpose(out, (0, 1, 3, 2))
